# Initial kernel scaffold; baseline (speedup 1.0000x reference)
import jax
import jax.numpy as jnp
from jax.experimental import pallas as pl


def kernel(vertices, edge_index, vertex_ids, bw1, bb1, bw2, bb2, nw1, nb1, nw2, nb2):
    raise NotImplementedError("write your pallas kernel here")


# speedup vs baseline: 5.3035x; 5.3035x over previous
import functools

import jax
import jax.numpy as jnp
from jax import lax
from jax.experimental import pallas as pl
from jax.experimental.pallas import tpu as pltpu
from jax.experimental.pallas import tpu_sc as plsc

N = 10000
D = 128
H_CONV = 128
H = 256
OUT = 128
E = 320000
B = 4096

NC = 2
NS = 16
TILES = NC * NS
L = 16
CHUNK = 128
GCH = 80
NBUF = 4
CPT = 80
GCPT = 128
IB = 8
EPT = CPT * CHUNK
EPAD = TILES * EPT
NPAD = 10240
ROWS_PT = NPAD // NS


def _fill_f32(ref, nrows, ncols, val):
    def row(i, carry):
        for k in range(ncols // L):
            ref[i, pl.ds(k * L, L)] = jnp.full((L,), val, jnp.float32)
        return carry
    lax.fori_loop(0, nrows, row, 0)


def _zero_acc_slice(zsrc_v, acc_sh, r0):
    for t in range(ROWS_PT // CHUNK):
        pltpu.sync_copy(zsrc_v, acc_sh.at[pl.ds(r0 + t * CHUNK, CHUNK)])


def _writeback_acc_slice(acc_sh, bounce_v, out_hbm, c, r0):
    for t in range(ROWS_PT // CHUNK):
        rr = r0 + t * CHUNK
        pltpu.sync_copy(acc_sh.at[pl.ds(rr, CHUNK)], bounce_v)
        pltpu.sync_copy(bounce_v, out_hbm.at[c, pl.ds(rr, CHUNK)])


def _edge_body(h_hbm, src_hbm, dst_hbm, s_out,
               src_v, dst_v, b0, b1, b2, b3, acc_sh, s0, s1, s2, s3):
    c = lax.axis_index("c")
    s = lax.axis_index("s")
    wid = c * NS + s
    r0 = s * ROWS_PT
    bufs = (b0, b1, b2, b3)
    sems = (s0, s1, s2, s3)

    _fill_f32(bufs[0], GCH, D, 0.0)
    for t in range(ROWS_PT // GCH):
        pltpu.sync_copy(bufs[0], acc_sh.at[pl.ds(r0 + t * GCH, GCH)])
    plsc.subcore_barrier()

    def step_batch(b, carry):
        pltpu.sync_copy(src_hbm.at[wid, pl.ds(b * IB, IB)], src_v)
        pltpu.sync_copy(dst_hbm.at[wid, pl.ds(b * IB, IB)], dst_v)
        cps = [None] * IB
        for a in range(NBUF - 1):
            cps[a] = pltpu.async_copy(h_hbm.at[dst_v.at[a]], bufs[a], sems[a])
        for jj in range(IB):
            cps[jj].wait()
            nxt = jj + NBUF - 1
            if nxt < IB:
                cps[nxt] = pltpu.async_copy(
                    h_hbm.at[dst_v.at[nxt]], bufs[nxt % NBUF], sems[nxt % NBUF])
            pltpu.sync_copy(bufs[jj % NBUF], acc_sh.at[src_v.at[jj]], add=True)
        return carry
    lax.fori_loop(0, GCPT // IB, step_batch, 0)

    plsc.subcore_barrier()
    for t in range(ROWS_PT // GCH):
        rr = r0 + t * GCH
        pltpu.sync_copy(acc_sh.at[pl.ds(rr, GCH)], bufs[0])
        pltpu.sync_copy(bufs[0], s_out.at[c, pl.ds(rr, GCH)])


_edge_kernel = pl.kernel(
    _edge_body,
    mesh=plsc.VectorSubcoreMesh(core_axis_name="c", subcore_axis_name="s"),
    out_type=[jax.ShapeDtypeStruct((NC, NPAD, D), jnp.float32)],
    scratch_types=[
        pltpu.VMEM((IB, GCH), jnp.int32),
        pltpu.VMEM((IB, GCH), jnp.int32),
        pltpu.VMEM((GCH, D), jnp.float32),
        pltpu.VMEM((GCH, D), jnp.float32),
        pltpu.VMEM((GCH, D), jnp.float32),
        pltpu.VMEM((GCH, D), jnp.float32),
        pltpu.VMEM_SHARED((NPAD, D), jnp.float32),
        pltpu.SemaphoreType.DMA,
        pltpu.SemaphoreType.DMA,
        pltpu.SemaphoreType.DMA,
        pltpu.SemaphoreType.DMA,
    ],
)


def _deg_body(src_hbm, d_out, src_v, ones_v, acc_sh, sem):
    c = lax.axis_index("c")
    s = lax.axis_index("s")
    wid = c * NS + s
    r0 = s * ROWS_PT

    _fill_f32(ones_v, CHUNK, D, 0.0)
    _zero_acc_slice(ones_v, acc_sh, r0)
    plsc.subcore_barrier()
    _fill_f32(ones_v, CHUNK, D, 1.0)

    def step_batch(b, carry):
        pltpu.sync_copy(src_hbm.at[wid, pl.ds(b * IB, IB)], src_v)
        for jj in range(IB):
            pltpu.sync_copy(ones_v, acc_sh.at[src_v.at[jj]], add=True)
        return carry
    lax.fori_loop(0, CPT // IB, step_batch, 0)

    plsc.subcore_barrier()
    _writeback_acc_slice(acc_sh, ones_v, d_out, c, r0)


_deg_kernel = pl.kernel(
    _deg_body,
    mesh=plsc.VectorSubcoreMesh(core_axis_name="c", subcore_axis_name="s"),
    out_type=[jax.ShapeDtypeStruct((NC, NPAD, D), jnp.float32)],
    scratch_types=[
        pltpu.VMEM((IB, CHUNK), jnp.int32),
        pltpu.VMEM((CHUNK, D), jnp.float32),
        pltpu.VMEM_SHARED((NPAD, D), jnp.float32),
        pltpu.SemaphoreType.DMA,
    ],
)


def _gather_body(h_hbm, ids_hbm, out_hbm, idx_v, rows_v, sem):
    c = lax.axis_index("c")
    s = lax.axis_index("s")
    wid = c * NS + s
    pltpu.sync_copy(ids_hbm.at[wid], idx_v)
    pltpu.async_copy(h_hbm.at[idx_v], rows_v, sem).wait()
    pltpu.sync_copy(rows_v, out_hbm.at[pl.ds(wid * CHUNK, CHUNK)])


_gather_kernel = pl.kernel(
    _gather_body,
    mesh=plsc.VectorSubcoreMesh(core_axis_name="c", subcore_axis_name="s"),
    out_type=[jax.ShapeDtypeStruct((B, D), jnp.float32)],
    scratch_types=[
        pltpu.VMEM((CHUNK,), jnp.int32),
        pltpu.VMEM((CHUNK, D), jnp.float32),
        pltpu.SemaphoreType.DMA,
    ],
)


def _elu(x):
    return jnp.where(x > 0, x, jnp.exp(x) - 1.0)


_RB = 1024


def _block_body(s_ref, d_ref, h_ref, w1_ref, b1_ref, w2_ref, b2_ref, out_ref):
    h = h_ref[...]
    ssum = s_ref[0] + s_ref[1]
    deg = 1.0 + d_ref[0, :, :1] + d_ref[1, :, :1]
    agg = (ssum + h) / deg
    z = jnp.dot(agg, w1_ref[...], preferred_element_type=jnp.float32) + b1_ref[...]
    z = _elu(z)
    z = jnp.dot(z, w2_ref[...], preferred_element_type=jnp.float32) + b2_ref[...] + h
    nrm = jnp.sqrt(jnp.sum(z * z, axis=1, keepdims=True))
    out_ref[...] = z / (nrm + 1e-12)


def _block_call(S, DG, h, w1, b1, w2, b2):
    return pl.pallas_call(
        _block_body,
        grid=(NPAD // _RB,),
        in_specs=[
            pl.BlockSpec((NC, _RB, D), lambda i: (0, i, 0)),
            pl.BlockSpec((NC, _RB, D), lambda i: (0, i, 0)),
            pl.BlockSpec((_RB, D), lambda i: (i, 0)),
            pl.BlockSpec((D, H_CONV), lambda i: (0, 0)),
            pl.BlockSpec((1, H_CONV), lambda i: (0, 0)),
            pl.BlockSpec((H_CONV, D), lambda i: (0, 0)),
            pl.BlockSpec((1, D), lambda i: (0, 0)),
        ],
        out_specs=pl.BlockSpec((_RB, D), lambda i: (i, 0)),
        out_shape=jax.ShapeDtypeStruct((NPAD, D), jnp.float32),
    )(S, DG, h, w1, b1, w2, b2)


_HB = 1024


def _head_body(hid_ref, w1_ref, b1_ref, w2_ref, b2_ref, out_ref):
    z = jnp.dot(hid_ref[...], w1_ref[...], preferred_element_type=jnp.float32) + b1_ref[...]
    z = _elu(z)
    out_ref[...] = jnp.dot(z, w2_ref[...], preferred_element_type=jnp.float32) + b2_ref[...]


def _head_call(hid, w1, b1, w2, b2):
    return pl.pallas_call(
        _head_body,
        grid=(B // _HB,),
        in_specs=[
            pl.BlockSpec((_HB, D), lambda i: (i, 0)),
            pl.BlockSpec((D, H), lambda i: (0, 0)),
            pl.BlockSpec((1, H), lambda i: (0, 0)),
            pl.BlockSpec((H, OUT), lambda i: (0, 0)),
            pl.BlockSpec((1, OUT), lambda i: (0, 0)),
        ],
        out_specs=pl.BlockSpec((_HB, OUT), lambda i: (i, 0)),
        out_shape=jax.ShapeDtypeStruct((B, OUT), jnp.float32),
    )(hid, w1, b1, w2, b2)


def kernel(vertices, edge_index, vertex_ids, bw1, bb1, bw2, bb2, nw1, nb1, nw2, nb2):
    src = edge_index[0].astype(jnp.int32)
    dst = edge_index[1].astype(jnp.int32)
    pad = EPAD - E
    src_flat = jnp.concatenate([src, jnp.full((pad,), N, jnp.int32)])
    dst_flat = jnp.concatenate([dst, jnp.zeros((pad,), jnp.int32)])
    src_p = src_flat.reshape(TILES, CPT, CHUNK)
    src_g = src_flat.reshape(TILES, GCPT, GCH)
    dst_g = dst_flat.reshape(TILES, GCPT, GCH)
    ids = vertex_ids.astype(jnp.int32).reshape(TILES, CHUNK)

    h = jnp.pad(vertices, ((0, NPAD - N), (0, 0)))
    (DG,) = _deg_kernel(src_p)
    for i in range(3):
        (S,) = _edge_kernel(h, src_g, dst_g)
        h = _block_call(S, DG, h, bw1[i], bb1[i].reshape(1, H_CONV),
                        bw2[i], bb2[i].reshape(1, D))
    (hid,) = _gather_kernel(h, ids)
    return _head_call(hid, nw1, nb1.reshape(1, H), nw2, nb2.reshape(1, OUT))
